# baseline (speedup 1.0000x reference)
import functools

import jax
import jax.numpy as jnp
from jax import lax
from jax.experimental import pallas as pl
from jax.experimental.pallas import tpu as pltpu
from jax.experimental.pallas import tpu_sc as plsc

B, H = 4096, 200
D = 128
N = B * H
NC, NS, L = 2, 16, 16
NW = NC * NS
BPW = B // NW
HH = H // 2

HALF_PI = 1.5707963267948966
PI = 3.141592653589793
A0 = 0.9998383860193922
A1 = -0.326983305517636
A2 = 0.15936586312036266
A3 = -0.047260694565070184
SIGN_MASK = -2147483648


@functools.partial(
    pl.kernel,
    out_type=jax.ShapeDtypeStruct((B, H, 2 * D), jnp.float32),
    mesh=plsc.VectorSubcoreMesh(core_axis_name="c", subcore_axis_name="s"),
    scratch_types=[
        pltpu.VMEM((2, HH), jnp.int32),
        pltpu.VMEM((H, D), jnp.float32),
        pltpu.VMEM((H, D), jnp.float32),
        pltpu.VMEM((H, 2 * D), jnp.float32),
        pltpu.SemaphoreType.DMA,
        pltpu.SemaphoreType.DMA,
    ],
)
def _qemb(idx_hbm, real_hbm, imag_hbm, out_hbm, idx_v, re_v, im_v, ob,
          sem_r, sem_i):
    wid = lax.axis_index("s") * NC + lax.axis_index("c")
    bbase = wid * BPW

    def chunk_body(bi, carry):
        gb = bbase + bi
        pltpu.sync_copy(idx_hbm.at[gb], idx_v)
        cr0 = pltpu.async_copy(
            real_hbm.at[idx_v.at[0]], re_v.at[pl.ds(0, HH)], sem_r)
        cr1 = pltpu.async_copy(
            real_hbm.at[idx_v.at[1]], re_v.at[pl.ds(HH, HH)], sem_r)
        ci0 = pltpu.async_copy(
            imag_hbm.at[idx_v.at[0]], im_v.at[pl.ds(0, HH)], sem_i)
        ci1 = pltpu.async_copy(
            imag_hbm.at[idx_v.at[1]], im_v.at[pl.ds(HH, HH)], sem_i)
        cr0.wait()
        cr1.wait()
        ci0.wait()
        ci1.wait()

        def row_body(row, c2):
            for l in range(D // L):
                sl = pl.ds(l * L, L)
                r = re_v[row, sl]
                i = im_v[row, sl]
                x = r * r + i * i
                xi = lax.bitcast_convert_type(x, jnp.int32)
                y = lax.bitcast_convert_type(
                    jnp.int32(0x5F3759DF) - (xi >> 1), jnp.float32)
                y = y * (1.5 - (0.5 * x) * (y * y))
                ax = jnp.abs(r)
                ay = jnp.abs(i)
                mx = jnp.maximum(ax, ay)
                mn = jnp.minimum(ax, ay)
                nz = mx > 0.0
                mag = jnp.where(nz, x * y, 0.0)
                den = jnp.where(nz, mx, 1.0)
                t = mn / den
                u = t * t
                p = A3
                p = p * u + A2
                p = p * u + A1
                p = p * u + A0
                ph = p * t
                ph = jnp.where(ay > ax, HALF_PI - ph, ph)
                ph = jnp.where(r < 0.0, PI - ph, ph)
                ph = lax.bitcast_convert_type(
                    lax.bitcast_convert_type(ph, jnp.int32)
                    ^ (lax.bitcast_convert_type(i, jnp.int32) & SIGN_MASK),
                    jnp.float32)
                ob[row, sl] = mag
                ob[row, pl.ds(D + l * L, L)] = ph
            return c2

        lax.fori_loop(0, H, row_body, 0, unroll=False)
        pltpu.sync_copy(ob, out_hbm.at[gb])
        return carry

    lax.fori_loop(0, BPW, chunk_body, 0, unroll=False)


def kernel(inputs, real_table, imag_table):
    idx = inputs.reshape(B, 2, HH).astype(jnp.int32)
    return _qemb(idx, real_table, imag_table)
